# baseline (speedup 1.0000x reference)
import functools

import jax
import jax.numpy as jnp
from jax import lax
from jax.experimental import pallas as pl
from jax.experimental.pallas import tpu as pltpu
from jax.experimental.pallas import tpu_sc as plsc

STREAM = 128
UNIT = 256
IPT = 512


@functools.lru_cache(maxsize=None)
def _make(W, N, D, DP):
    info = plsc.get_sparse_core_info()
    NC, NS = info.num_cores, info.num_subcores
    NW = NC * NS
    assert N == NW * IPT and IPT % UNIT == 0 and UNIT % STREAM == 0
    units_per_w = IPT // UNIT
    n_units = W * units_per_w
    mesh = plsc.VectorSubcoreMesh(core_axis_name="c", subcore_axis_name="s")

    @functools.partial(
        pl.kernel,
        mesh=mesh,
        compiler_params=pltpu.CompilerParams(
            needs_layout_passes=False, use_tc_tiling_on_sc=True),
        out_type=jax.ShapeDtypeStruct((W, D, N), jnp.float32),
        scratch_types=[
            pltpu.VMEM((W, IPT), jnp.int32),
            pltpu.VMEM((UNIT, DP), jnp.float32),
            pltpu.VMEM((UNIT, DP), jnp.float32),
            pltpu.VMEM((D, UNIT), jnp.float32),
            pltpu.VMEM((D, UNIT), jnp.float32),
            pltpu.SemaphoreType.DMA,
            pltpu.SemaphoreType.DMA,
            pltpu.SemaphoreType.DMA,
            pltpu.SemaphoreType.DMA,
        ],
    )
    def k(nodesT_hbm, tpad_hbm, out_hbm, idx_v, rb0, rb1, tb0, tb1,
          gsem0, gsem1, wsem0, wsem1):
        wid = lax.axis_index("s") * NC + lax.axis_index("c")
        i0 = wid * IPT
        pltpu.sync_copy(nodesT_hbm.at[:, pl.ds(i0, IPT)], idx_v)
        lanes = lax.iota(jnp.int32, 16)

        def fire(u, rb, gsem):
            w = u // units_per_w
            col = (u % units_per_w) * UNIT
            return [
                pltpu.async_copy(
                    tpad_hbm.at[idx_v.at[w, pl.ds(col + s * STREAM, STREAM)]],
                    rb.at[pl.ds(s * STREAM, STREAM)], gsem)
                for s in range(UNIT // STREAM)
            ]

        def transpose(rb, tb):
            def body(j, carry):
                rows = j * 16 + lanes
                for d in range(D):
                    dv = jnp.full((16,), d, jnp.int32)
                    vals = plsc.load_gather(rb, [rows, dv])
                    plsc.store_scatter(tb, [dv, rows], vals)
                return carry
            lax.fori_loop(0, UNIT // 16, body, 0)

        def wstart(u, tb, wsem):
            w = u // units_per_w
            col = i0 + (u % units_per_w) * UNIT
            return pltpu.async_copy(tb, out_hbm.at[w, :, pl.ds(col, UNIT)], wsem)

        def body(t, carry):
            e = 2 * t
            o = e + 1
            ge = fire(e, rb0, gsem0)
            go = fire(o, rb1, gsem1)
            for h in ge:
                h.wait()
            transpose(rb0, tb0)
            we = wstart(e, tb0, wsem0)
            for h in go:
                h.wait()
            transpose(rb1, tb1)
            wo = wstart(o, tb1, wsem1)
            we.wait()
            wo.wait()
            return carry

        lax.fori_loop(0, n_units // 2, body, 0)

    return k


def kernel(nodes, table):
    n, w = nodes.shape
    v, d = table.shape
    nodesT = nodes.T.astype(jnp.int32)
    tpad = jnp.pad(table, ((0, 0), (0, d)))
    outT = _make(w, n, d, 2 * d)(nodesT, tpad)
    return outT.transpose(2, 0, 1)
